# baseline (speedup 1.0000x reference)
import numpy as np

import jax
import jax.numpy as jnp
from jax import lax
from jax.experimental import pallas as pl
from jax.experimental.pallas import tpu as pltpu
from jax.experimental.pallas import tpu_sc as plsc

N_NODES = 50000
N_EDGES = 800000
N_SPECIES = 4
N_PSEUDO = 2
N_MAX = 4
R_CUT = 5.0

F = 72
FP = 128
NSH = 9
NSP = NSH * N_PSEUDO
BAS = 24
E_PAD = 802816
N_GROUPS = E_PAD // 128
TC_BLK = 128
TC_GRID = N_GROUPS // TC_BLK

NC = 2
NS = 16
NPASS = 2
CHUNK = 80
EDGES_PER_TILE = E_PAD // NS
ROUNDS = 8
RND_EDGES = EDGES_PER_TILE // ROUNDS
QUARTER = 12504
ACC_ROWS = 12544
SPAN = ACC_ROWS // NS
TRASH = QUARTER
ZCHUNK = 8

_C1 = 0.4886025119029199
_C2A = 1.0925484305920792
_C2B = 0.31539156525252005
_C2C = 0.5462742152960396
_SH0 = 0.28209479177387814


def _selection_matrices():
    k1 = np.zeros((BAS, FP), np.float32)
    k2 = np.zeros((BAS, FP), np.float32)
    for i in range(NSP):
        for n in range(N_MAX):
            k1[i, i * N_MAX + n] = 1.0
            k2[NSP + n, i * N_MAX + n] = 1.0
    return jnp.asarray(k1), jnp.asarray(k2)


def _tc_feat_kernel(xs, ys, zs, sp, comb, k1, k2, feat, scr):
    x = xs[...]
    y = ys[...]
    z = zs[...]
    s2 = x * x + y * y + z * z + 1e-12
    r = jnp.sqrt(s2)
    inv = 1.0 / r
    ux = x * inv
    uy = y * inv
    uz = z * inv

    sh = [
        jnp.full(x.shape, _SH0, dtype=x.dtype),
        _C1 * uy,
        _C1 * uz,
        _C1 * ux,
        _C2A * ux * uy,
        _C2A * uy * uz,
        _C2B * (3.0 * uz * uz - 1.0),
        _C2A * ux * uz,
        _C2C * (ux * ux - uy * uy),
    ]

    spec = sp[...]
    p = []
    for pp in range(N_PSEUDO):
        acc = jnp.zeros(x.shape, dtype=x.dtype)
        for ss in range(N_SPECIES):
            acc = acc + jnp.where(spec == ss, comb[pp, ss], 0.0)
        p.append(acc)

    gamma = (N_MAX / R_CUT) ** 2
    fc = jnp.where(r < R_CUT, 0.5 * (jnp.cos(jnp.pi * r / R_CUT) + 1.0), 0.0)
    for lm in range(NSH):
        for pp in range(N_PSEUDO):
            scr[:, lm * N_PSEUDO + pp, :] = sh[lm] * p[pp]
    for n in range(N_MAX):
        d = r - R_CUT * n / (N_MAX - 1)
        scr[:, NSP + n, :] = jnp.exp(-gamma * d * d) * fc

    k1v = k1[...]
    k2v = k2[...]
    dims = (((0,), (0,)), ((), ()))
    for s in range(TC_BLK):
        m = scr[s]
        v1 = lax.dot_general(m, k1v, dims, preferred_element_type=jnp.float32)
        v2 = lax.dot_general(m, k2v, dims, preferred_element_type=jnp.float32)
        feat[pl.ds(s * 128, 128), :] = v1 * v2


def _tc_feat(xs, ys, zs, sp, comb_pad, k1, k2):
    return pl.pallas_call(
        _tc_feat_kernel,
        grid=(TC_GRID,),
        in_specs=[
            pl.BlockSpec((TC_BLK, 128), lambda g: (g, 0)),
            pl.BlockSpec((TC_BLK, 128), lambda g: (g, 0)),
            pl.BlockSpec((TC_BLK, 128), lambda g: (g, 0)),
            pl.BlockSpec((TC_BLK, 128), lambda g: (g, 0)),
            pl.BlockSpec((8, 128), lambda g: (0, 0)),
            pl.BlockSpec((BAS, FP), lambda g: (0, 0)),
            pl.BlockSpec((BAS, FP), lambda g: (0, 0)),
        ],
        out_specs=pl.BlockSpec((TC_BLK * 128, FP), lambda g: (g, 0)),
        out_shape=jax.ShapeDtypeStruct((E_PAD, FP), jnp.float32),
        scratch_shapes=[pltpu.VMEM((TC_BLK, BAS, 128), jnp.float32)],
    )(xs, ys, zs, sp, comb_pad, k1, k2)


def _sc_body(
    feat_hbm, cent_hbm, zeros_hbm, out_hbm,
    acc, rows0, rows1, pk_buf, eidb0, eidb1, idxb0, idxb1, zb,
    g_sem0, g_sem1, sc_sem0, sc_sem1,
):
    c = lax.axis_index("c")
    s = lax.axis_index("s")
    tile_e0 = s * EDGES_PER_TILE
    rows = (rows0, rows1)
    eidb = (eidb0, eidb1)
    idxb = (idxb0, idxb1)
    g_sem = (g_sem0, g_sem1)
    sc_sem = (sc_sem0, sc_sem1)
    iota16 = lax.iota(jnp.int32, 16)

    def start_gather(slot):
        pltpu.async_copy(feat_hbm.at[eidb[slot]], rows[slot], g_sem[slot])

    def wait_gather(slot):
        pltpu.make_async_copy(feat_hbm.at[eidb[slot]], rows[slot], g_sem[slot]).wait()

    def start_scatter(slot):
        pltpu.async_copy(rows[slot], acc.at[idxb[slot]], sc_sem[slot], add=True)

    def wait_scatter(slot):
        pltpu.make_async_copy(rows[slot], acc.at[idxb[slot]], sc_sem[slot]).wait()

    for p in range(NPASS):
        base = p * (NC * QUARTER) + c * QUARTER

        pltpu.sync_copy(zeros_hbm, zb)
        for t in range(SPAN // ZCHUNK):
            pltpu.sync_copy(zb, acc.at[pl.ds(s * SPAN + t * ZCHUNK, ZCHUNK)])
        plsc.subcore_barrier()

        for rnd in range(ROUNDS):
            round_e0 = tile_e0 + rnd * RND_EDGES

            pltpu.sync_copy(cent_hbm.at[pl.ds(round_e0, RND_EDGES)], pk_buf)

            def compact_body(k, ptr_vec):
                for j in range(8):
                    off = k * 128 + j * 16
                    cv = pk_buf[pl.ds(off, 16)]
                    li = cv - base
                    ok = (li >= 0) & (li < QUARTER)
                    packed = jnp.where(ok, (li << 16) | (off + iota16), 0)
                    oki = ok.astype(jnp.int32)
                    cum = plsc.cumsum(oki)
                    pos = ptr_vec + cum - oki
                    plsc.store_scatter(pk_buf, [pos], packed, mask=ok)
                    ptr_vec = ptr_vec + plsc.all_reduce_population_count(ok)
                return ptr_vec

            ptr_vec = lax.fori_loop(
                0, RND_EDGES // 128, compact_body, jnp.zeros((16,), jnp.int32)
            )
            n_hits = ptr_vec[0]

            def build(b, slot):
                for j in range(CHUNK // 16):
                    pos = b * CHUNK + j * 16
                    rpos = jnp.minimum(pos, RND_EDGES - 16)
                    pk = pk_buf[pl.ds(rpos, 16)]
                    valid = (pos + iota16) < n_hits
                    eidb[slot][pl.ds(j * 16, 16)] = jnp.where(
                        valid, (pk & 0xFFFF) + round_e0, round_e0
                    )
                    idxb[slot][pl.ds(j * 16, 16)] = jnp.where(
                        valid, lax.shift_right_logical(pk, 16), TRASH
                    )

            nb = (n_hits + (CHUNK - 1)) // CHUNK
            nb2 = jnp.maximum((nb + 1) // 2, 1)

            build(0, 0)
            start_gather(0)
            build(1, 1)
            start_gather(1)

            def batch_body(kk, carry):
                b0 = kk * 2
                wait_gather(0)
                start_scatter(0)
                wait_scatter(0)
                build(b0 + 2, 0)
                start_gather(0)
                wait_gather(1)
                start_scatter(1)
                wait_scatter(1)
                build(b0 + 3, 1)
                start_gather(1)
                return carry

            lax.fori_loop(0, nb2 - 1, batch_body, 0)
            for slot in (0, 1):
                wait_gather(slot)
                start_scatter(slot)
                wait_scatter(slot)

        plsc.subcore_barrier()
        pltpu.sync_copy(
            acc.at[pl.ds(s * SPAN, SPAN)],
            out_hbm.at[c, p, pl.ds(s * SPAN, SPAN)],
        )
        plsc.subcore_barrier()


def _sc_scatter(feat, cent1d, zeros_hbm):
    mesh = plsc.VectorSubcoreMesh(
        core_axis_name="c", subcore_axis_name="s", num_cores=NC, num_subcores=NS
    )
    return pl.kernel(
        _sc_body,
        out_type=jax.ShapeDtypeStruct((NC, NPASS, ACC_ROWS, FP), jnp.float32),
        mesh=mesh,
        compiler_params=pltpu.CompilerParams(needs_layout_passes=False),
        scratch_types=[
            pltpu.VMEM_SHARED((ACC_ROWS, FP), jnp.float32),
            pltpu.VMEM((CHUNK, FP), jnp.float32),
            pltpu.VMEM((CHUNK, FP), jnp.float32),
            pltpu.VMEM((RND_EDGES,), jnp.int32),
            pltpu.VMEM((CHUNK,), jnp.int32),
            pltpu.VMEM((CHUNK,), jnp.int32),
            pltpu.VMEM((CHUNK,), jnp.int32),
            pltpu.VMEM((CHUNK,), jnp.int32),
            pltpu.VMEM((ZCHUNK, FP), jnp.float32),
            pltpu.SemaphoreType.DMA,
            pltpu.SemaphoreType.DMA,
            pltpu.SemaphoreType.DMA,
            pltpu.SemaphoreType.DMA,
        ],
    )(feat, cent1d, zeros_hbm)


@jax.jit
def kernel(edge_vectors, centers, neighbor_species, combination_matrix):
    ev = jnp.pad(edge_vectors, ((0, E_PAD - N_EDGES), (0, 0)))
    xs = ev[:, 0].reshape(N_GROUPS, 128)
    ys = ev[:, 1].reshape(N_GROUPS, 128)
    zs = ev[:, 2].reshape(N_GROUPS, 128)
    sp = jnp.pad(neighbor_species.astype(jnp.int32), (0, E_PAD - N_EDGES)).reshape(
        N_GROUPS, 128
    )
    cent1d = jnp.pad(
        centers.astype(jnp.int32), (0, E_PAD - N_EDGES), constant_values=N_NODES
    )
    comb_pad = jnp.zeros((8, 128), jnp.float32).at[:N_PSEUDO, :N_SPECIES].set(
        combination_matrix.astype(jnp.float32)
    )
    zeros_hbm = jnp.zeros((ZCHUNK, FP), jnp.float32)
    k1, k2 = _selection_matrices()

    feat = _tc_feat(xs, ys, zs, sp, comb_pad, k1, k2)
    out4 = _sc_scatter(feat, cent1d, zeros_hbm)
    full = jnp.concatenate(
        [
            out4[0, 0, :QUARTER],
            out4[1, 0, :QUARTER],
            out4[0, 1, :QUARTER],
            out4[1, 1, :QUARTER],
        ],
        axis=0,
    )
    return full[:N_NODES, :F]


# speedup vs baseline: 1.0945x; 1.0945x over previous
import numpy as np

import jax
import jax.numpy as jnp
from jax import lax
from jax.experimental import pallas as pl
from jax.experimental.pallas import tpu as pltpu
from jax.experimental.pallas import tpu_sc as plsc

N_NODES = 50000
N_EDGES = 800000
N_SPECIES = 4
N_PSEUDO = 2
N_MAX = 4
R_CUT = 5.0

F = 72
FP = 128
NSH = 9
NSP = NSH * N_PSEUDO
BAS = 24
E_PAD = 802816
N_GROUPS = E_PAD // 128
TC_BLK = 128
TC_GRID = N_GROUPS // TC_BLK

NC = 2
NS = 16
NPASS = 2
CHUNK = 64
EDGES_PER_TILE = E_PAD // NS
ROUNDS = 4
RND_EDGES = EDGES_PER_TILE // ROUNDS
QUARTER = 12504
ACC_ROWS = 12544
SPAN = ACC_ROWS // NS
TRASH = QUARTER
ZCHUNK = 8

_C1 = 0.4886025119029199
_C2A = 1.0925484305920792
_C2B = 0.31539156525252005
_C2C = 0.5462742152960396
_SH0 = 0.28209479177387814


def _selection_matrices():
    k1 = np.zeros((BAS, FP), np.float32)
    k2 = np.zeros((BAS, FP), np.float32)
    for i in range(NSP):
        for n in range(N_MAX):
            k1[i, i * N_MAX + n] = 1.0
            k2[NSP + n, i * N_MAX + n] = 1.0
    return jnp.asarray(k1), jnp.asarray(k2)


def _tc_feat_kernel(xs, ys, zs, sp, comb, k1, k2, feat, scr):
    x = xs[...]
    y = ys[...]
    z = zs[...]
    s2 = x * x + y * y + z * z + 1e-12
    r = jnp.sqrt(s2)
    inv = 1.0 / r
    ux = x * inv
    uy = y * inv
    uz = z * inv

    sh = [
        jnp.full(x.shape, _SH0, dtype=x.dtype),
        _C1 * uy,
        _C1 * uz,
        _C1 * ux,
        _C2A * ux * uy,
        _C2A * uy * uz,
        _C2B * (3.0 * uz * uz - 1.0),
        _C2A * ux * uz,
        _C2C * (ux * ux - uy * uy),
    ]

    spec = sp[...]
    p = []
    for pp in range(N_PSEUDO):
        acc = jnp.zeros(x.shape, dtype=x.dtype)
        for ss in range(N_SPECIES):
            acc = acc + jnp.where(spec == ss, comb[pp, ss], 0.0)
        p.append(acc)

    gamma = (N_MAX / R_CUT) ** 2
    fc = jnp.where(r < R_CUT, 0.5 * (jnp.cos(jnp.pi * r / R_CUT) + 1.0), 0.0)
    for lm in range(NSH):
        for pp in range(N_PSEUDO):
            scr[:, lm * N_PSEUDO + pp, :] = sh[lm] * p[pp]
    for n in range(N_MAX):
        d = r - R_CUT * n / (N_MAX - 1)
        scr[:, NSP + n, :] = jnp.exp(-gamma * d * d) * fc

    k1v = k1[...]
    k2v = k2[...]
    dims = (((0,), (0,)), ((), ()))
    for s in range(TC_BLK):
        m = scr[s]
        v1 = lax.dot_general(m, k1v, dims, preferred_element_type=jnp.float32)
        v2 = lax.dot_general(m, k2v, dims, preferred_element_type=jnp.float32)
        feat[pl.ds(s * 128, 128), :] = v1 * v2


def _tc_feat(xs, ys, zs, sp, comb_pad, k1, k2):
    return pl.pallas_call(
        _tc_feat_kernel,
        grid=(TC_GRID,),
        in_specs=[
            pl.BlockSpec((TC_BLK, 128), lambda g: (g, 0)),
            pl.BlockSpec((TC_BLK, 128), lambda g: (g, 0)),
            pl.BlockSpec((TC_BLK, 128), lambda g: (g, 0)),
            pl.BlockSpec((TC_BLK, 128), lambda g: (g, 0)),
            pl.BlockSpec((8, 128), lambda g: (0, 0)),
            pl.BlockSpec((BAS, FP), lambda g: (0, 0)),
            pl.BlockSpec((BAS, FP), lambda g: (0, 0)),
        ],
        out_specs=pl.BlockSpec((TC_BLK * 128, FP), lambda g: (g, 0)),
        out_shape=jax.ShapeDtypeStruct((E_PAD, FP), jnp.float32),
        scratch_shapes=[pltpu.VMEM((TC_BLK, BAS, 128), jnp.float32)],
    )(xs, ys, zs, sp, comb_pad, k1, k2)


def _sc_body(
    feat_hbm, cent_hbm, zeros_hbm, out_hbm,
    acc, rows0, rows1, pk_buf, eidb0, eidb1, idxb0, idxb1, zb,
    g_sem0, g_sem1, sc_sem0, sc_sem1,
):
    c = lax.axis_index("c")
    s = lax.axis_index("s")
    tile_e0 = s * EDGES_PER_TILE
    rows = (rows0, rows1)
    eidb = (eidb0, eidb1)
    idxb = (idxb0, idxb1)
    g_sem = (g_sem0, g_sem1)
    sc_sem = (sc_sem0, sc_sem1)
    iota16 = lax.iota(jnp.int32, 16)

    def start_gather(slot):
        pltpu.async_copy(feat_hbm.at[eidb[slot]], rows[slot], g_sem[slot])

    def wait_gather(slot):
        pltpu.make_async_copy(feat_hbm.at[eidb[slot]], rows[slot], g_sem[slot]).wait()

    def start_scatter(slot):
        pltpu.async_copy(rows[slot], acc.at[idxb[slot]], sc_sem[slot], add=True)

    def wait_scatter(slot):
        pltpu.make_async_copy(rows[slot], acc.at[idxb[slot]], sc_sem[slot]).wait()

    for p in range(NPASS):
        base = p * (NC * QUARTER) + c * QUARTER

        pltpu.sync_copy(zeros_hbm, zb)
        for t in range(SPAN // ZCHUNK):
            pltpu.sync_copy(zb, acc.at[pl.ds(s * SPAN + t * ZCHUNK, ZCHUNK)])
        plsc.subcore_barrier()

        for rnd in range(ROUNDS):
            round_e0 = tile_e0 + rnd * RND_EDGES

            pltpu.sync_copy(cent_hbm.at[pl.ds(round_e0, RND_EDGES)], pk_buf)

            def compact_body(k, ptr_vec):
                for j in range(8):
                    off = k * 128 + j * 16
                    cv = pk_buf[pl.ds(off, 16)]
                    li = cv - base
                    ok = (li >= 0) & (li < QUARTER)
                    packed = jnp.where(ok, (li << 16) | (off + iota16), 0)
                    oki = ok.astype(jnp.int32)
                    cum = plsc.cumsum(oki)
                    pos = ptr_vec + cum - oki
                    plsc.store_scatter(pk_buf, [pos], packed, mask=ok)
                    ptr_vec = ptr_vec + plsc.all_reduce_population_count(ok)
                return ptr_vec

            ptr_vec = lax.fori_loop(
                0, RND_EDGES // 128, compact_body, jnp.zeros((16,), jnp.int32)
            )
            n_hits = ptr_vec[0]

            def build(b, slot):
                for j in range(CHUNK // 16):
                    pos = b * CHUNK + j * 16
                    rpos = jnp.minimum(pos, RND_EDGES - 16)
                    pk = pk_buf[pl.ds(rpos, 16)]
                    valid = (pos + iota16) < n_hits
                    eidb[slot][pl.ds(j * 16, 16)] = jnp.where(
                        valid, (pk & 0xFFFF) + round_e0, round_e0
                    )
                    idxb[slot][pl.ds(j * 16, 16)] = jnp.where(
                        valid, lax.shift_right_logical(pk, 16), TRASH
                    )

            nb = (n_hits + (CHUNK - 1)) // CHUNK
            nb2 = jnp.maximum((nb + 1) // 2, 1)

            build(0, 0)
            start_gather(0)
            build(1, 1)
            start_gather(1)

            def batch_body(kk, carry):
                b0 = kk * 2
                wait_gather(0)
                start_scatter(0)
                wait_scatter(0)
                build(b0 + 2, 0)
                start_gather(0)
                wait_gather(1)
                start_scatter(1)
                wait_scatter(1)
                build(b0 + 3, 1)
                start_gather(1)
                return carry

            lax.fori_loop(0, nb2 - 1, batch_body, 0)
            for slot in (0, 1):
                wait_gather(slot)
                start_scatter(slot)
                wait_scatter(slot)

        plsc.subcore_barrier()
        pltpu.sync_copy(
            acc.at[pl.ds(s * SPAN, SPAN)],
            out_hbm.at[c, p, pl.ds(s * SPAN, SPAN)],
        )
        plsc.subcore_barrier()


def _sc_scatter(feat, cent1d, zeros_hbm):
    mesh = plsc.VectorSubcoreMesh(
        core_axis_name="c", subcore_axis_name="s", num_cores=NC, num_subcores=NS
    )
    return pl.kernel(
        _sc_body,
        out_type=jax.ShapeDtypeStruct((NC, NPASS, ACC_ROWS, FP), jnp.float32),
        mesh=mesh,
        compiler_params=pltpu.CompilerParams(needs_layout_passes=False),
        scratch_types=[
            pltpu.VMEM_SHARED((ACC_ROWS, FP), jnp.float32),
            pltpu.VMEM((CHUNK, FP), jnp.float32),
            pltpu.VMEM((CHUNK, FP), jnp.float32),
            pltpu.VMEM((RND_EDGES,), jnp.int32),
            pltpu.VMEM((CHUNK,), jnp.int32),
            pltpu.VMEM((CHUNK,), jnp.int32),
            pltpu.VMEM((CHUNK,), jnp.int32),
            pltpu.VMEM((CHUNK,), jnp.int32),
            pltpu.VMEM((ZCHUNK, FP), jnp.float32),
            pltpu.SemaphoreType.DMA,
            pltpu.SemaphoreType.DMA,
            pltpu.SemaphoreType.DMA,
            pltpu.SemaphoreType.DMA,
        ],
    )(feat, cent1d, zeros_hbm)


@jax.jit
def kernel(edge_vectors, centers, neighbor_species, combination_matrix):
    ev = jnp.pad(edge_vectors, ((0, E_PAD - N_EDGES), (0, 0)))
    xs = ev[:, 0].reshape(N_GROUPS, 128)
    ys = ev[:, 1].reshape(N_GROUPS, 128)
    zs = ev[:, 2].reshape(N_GROUPS, 128)
    sp = jnp.pad(neighbor_species.astype(jnp.int32), (0, E_PAD - N_EDGES)).reshape(
        N_GROUPS, 128
    )
    cent1d = jnp.pad(
        centers.astype(jnp.int32), (0, E_PAD - N_EDGES), constant_values=N_NODES
    )
    comb_pad = jnp.zeros((8, 128), jnp.float32).at[:N_PSEUDO, :N_SPECIES].set(
        combination_matrix.astype(jnp.float32)
    )
    zeros_hbm = jnp.zeros((ZCHUNK, FP), jnp.float32)
    k1, k2 = _selection_matrices()

    feat = _tc_feat(xs, ys, zs, sp, comb_pad, k1, k2)
    out4 = _sc_scatter(feat, cent1d, zeros_hbm)
    full = jnp.concatenate(
        [
            out4[0, 0, :QUARTER],
            out4[1, 0, :QUARTER],
            out4[0, 1, :QUARTER],
            out4[1, 1, :QUARTER],
        ],
        axis=0,
    )
    return full[:N_NODES, :F]
